# baseline (speedup 1.0000x reference)
import functools
import math

import jax
import jax.numpy as jnp
from jax import lax
from jax.experimental import pallas as pl
from jax.experimental.pallas import tpu as pltpu
from jax.experimental.pallas import tpu_sc as plsc

DIM = 64
SCALE = math.sqrt(DIM)
NUM_CORES = 2
NUM_SUBCORES = 16
NUM_WORKERS = NUM_CORES * NUM_SUBCORES
LANES = 16

SEQ = 200
GATHER_SUBS = (128, 72)


def _emb_kernel(num_seqs):
    s_per_w = num_seqs // NUM_WORKERS
    t_per_w = s_per_w * SEQ
    mesh = plsc.VectorSubcoreMesh(core_axis_name="c", subcore_axis_name="s")

    @functools.partial(
        pl.kernel,
        mesh=mesh,
        out_type=jax.ShapeDtypeStruct((num_seqs, SEQ, DIM), jnp.float32),
        scratch_types=[
            pltpu.VMEM((t_per_w,), jnp.int32),
            pltpu.VMEM((SEQ,), jnp.int32),
            pltpu.VMEM((SEQ,), jnp.int32),
            pltpu.VMEM((SEQ, 2 * DIM), jnp.float32),
            pltpu.VMEM((SEQ, 2 * DIM), jnp.float32),
            pltpu.VMEM((SEQ, DIM), jnp.float32),
            pltpu.VMEM((SEQ, DIM), jnp.float32),
            pltpu.SemaphoreType.DMA,
            pltpu.SemaphoreType.DMA,
            pltpu.SemaphoreType.DMA,
            pltpu.SemaphoreType.DMA,
        ],
        compiler_params=pltpu.CompilerParams(use_tc_tiling_on_sc=True),
    )
    def body(idxp_hbm, cb_hbm, pairs_hbm, out_hbm,
             idxp_v, cb_a, cb_b, rows_a, rows_b, out_a, out_b,
             gsem_a, gsem_b, ssem_a, ssem_b):
        wid = lax.axis_index("s") * NUM_CORES + lax.axis_index("c")
        sbase = wid * s_per_w
        cbufs = (cb_a, cb_b)
        rbufs = (rows_a, rows_b)
        obufs = (out_a, out_b)
        gsems = (gsem_a, gsem_b)
        ssems = (ssem_a, ssem_b)

        tbase = pl.multiple_of(sbase * SEQ, 8)
        pltpu.sync_copy(idxp_hbm.at[pl.ds(tbase, t_per_w)], idxp_v)

        def fire_gather(c, b):
            off = 0
            for n in GATHER_SUBS:
                pltpu.async_copy(
                    pairs_hbm.at[idxp_v.at[pl.ds(c * SEQ + off, n)]],
                    rbufs[b].at[pl.ds(off, n)],
                    gsems[b],
                )
                off += n

        def drain_gather(b):
            off = 0
            for n in GATHER_SUBS:
                pltpu.make_async_copy(
                    pairs_hbm.at[idxp_v.at[pl.ds(off, n)]],
                    rbufs[b].at[pl.ds(off, n)],
                    gsems[b],
                ).wait()
                off += n

        def load_cb(c, b):
            off = pl.multiple_of((sbase + c) * SEQ, 8)
            pltpu.sync_copy(cb_hbm.at[pl.ds(off, SEQ)], cbufs[b])

        def wait_store(b):
            pltpu.make_async_copy(
                obufs[b], out_hbm.at[sbase], ssems[b]
            ).wait()

        load_cb(0, 0)
        fire_gather(0, 0)

        def step(c, b):
            rows, outb = rbufs[b], obufs[b]

            @pl.when(c >= 2)
            def _():
                wait_store(b)

            @pl.when(c + 1 < s_per_w)
            def _():
                fire_gather(c + 1, 1 - b)
                load_cb(c + 1, 1 - b)

            drain_gather(b)

            def extract_16(tbase16):
                cb16 = cbufs[b][pl.ds(tbase16, LANES)]
                for j in range(LANES):
                    t = tbase16 + j
                    cb = cb16[j]
                    for g in range(DIM // LANES):
                        outb[t, pl.ds(g * LANES, LANES)] = (
                            rows[t, pl.ds(cb + g * LANES, LANES)]
                        )

            def extract_group(q, carry):
                extract_16(q * LANES)
                return carry

            lax.fori_loop(0, SEQ // LANES, extract_group, 0)
            extract_16(SEQ - LANES)

            pltpu.async_copy(outb, out_hbm.at[sbase + c], ssems[b])

        def pair_steps(p, carry):
            step(2 * p, 0)
            step(2 * p + 1, 1)
            return carry

        lax.fori_loop(0, s_per_w // 2, pair_steps, 0)
        wait_store((s_per_w - 1) % 2)

    return body


def kernel(x, table):
    num_seqs = x.shape[0]
    idx = jnp.reshape(x, (x.size,)).astype(jnp.int32)
    idxp = idx >> 1
    cb = (idx & 1) << 6
    pairs = jnp.reshape(table * SCALE, (table.shape[0] // 2, 2 * DIM))
    return _emb_kernel(num_seqs)(idxp, cb, pairs)


# speedup vs baseline: 1.0400x; 1.0400x over previous
import functools
import math

import jax
import jax.numpy as jnp
from jax import lax
from jax.experimental import pallas as pl
from jax.experimental.pallas import tpu as pltpu
from jax.experimental.pallas import tpu_sc as plsc

DIM = 64
SCALE = math.sqrt(DIM)
NUM_CORES = 2
NUM_SUBCORES = 16
NUM_WORKERS = NUM_CORES * NUM_SUBCORES
LANES = 16

SEQ = 200
GATHER_SUBS = (128, 72)


def _emb_kernel(num_seqs):
    s_per_w = num_seqs // NUM_WORKERS
    t_per_w = s_per_w * SEQ
    mesh = plsc.VectorSubcoreMesh(core_axis_name="c", subcore_axis_name="s")

    @functools.partial(
        pl.kernel,
        mesh=mesh,
        out_type=jax.ShapeDtypeStruct((num_seqs, SEQ, DIM), jnp.float32),
        scratch_types=[
            pltpu.VMEM((t_per_w,), jnp.int32),
            pltpu.VMEM((SEQ,), jnp.int32),
            pltpu.VMEM((SEQ,), jnp.int32),
            pltpu.VMEM((SEQ, 2 * DIM), jnp.float32),
            pltpu.VMEM((SEQ, 2 * DIM), jnp.float32),
            pltpu.VMEM((SEQ, DIM), jnp.float32),
            pltpu.VMEM((SEQ, DIM), jnp.float32),
            pltpu.SemaphoreType.DMA,
            pltpu.SemaphoreType.DMA,
            pltpu.SemaphoreType.DMA,
            pltpu.SemaphoreType.DMA,
        ],
        compiler_params=pltpu.CompilerParams(use_tc_tiling_on_sc=True),
    )
    def body(idxp_hbm, cb_hbm, pairs_hbm, out_hbm,
             idxp_v, cb_a, cb_b, rows_a, rows_b, out_a, out_b,
             gsem_a, gsem_b, ssem_a, ssem_b):
        wid = lax.axis_index("s") * NUM_CORES + lax.axis_index("c")
        sbase = wid * s_per_w
        cbufs = (cb_a, cb_b)
        rbufs = (rows_a, rows_b)
        obufs = (out_a, out_b)
        gsems = (gsem_a, gsem_b)
        ssems = (ssem_a, ssem_b)

        tbase = pl.multiple_of(sbase * SEQ, 8)
        pltpu.sync_copy(idxp_hbm.at[pl.ds(tbase, t_per_w)], idxp_v)

        def fire_gather(c, b):
            off = 0
            for n in GATHER_SUBS:
                pltpu.async_copy(
                    pairs_hbm.at[idxp_v.at[pl.ds(c * SEQ + off, n)]],
                    rbufs[b].at[pl.ds(off, n)],
                    gsems[b],
                )
                off += n

        def drain_gather(b):
            off = 0
            for n in GATHER_SUBS:
                pltpu.make_async_copy(
                    pairs_hbm.at[idxp_v.at[pl.ds(off, n)]],
                    rbufs[b].at[pl.ds(off, n)],
                    gsems[b],
                ).wait()
                off += n

        def load_cb(c, b):
            off = pl.multiple_of((sbase + c) * SEQ, 8)
            pltpu.sync_copy(cb_hbm.at[pl.ds(off, SEQ)], cbufs[b])

        def wait_store(b):
            pltpu.make_async_copy(
                obufs[b], out_hbm.at[sbase], ssems[b]
            ).wait()

        load_cb(0, 0)
        fire_gather(0, 0)

        def step(c, b):
            rows, outb = rbufs[b], obufs[b]

            @pl.when(c >= 2)
            def _():
                wait_store(b)

            @pl.when(c + 1 < s_per_w)
            def _():
                fire_gather(c + 1, 1 - b)
                load_cb(c + 1, 1 - b)

            drain_gather(b)

            def extract_16(tbase16):
                cb16 = cbufs[b][pl.ds(tbase16, LANES)]
                for j in range(LANES):
                    t = tbase16 + j
                    cb = cb16[j]
                    for g in range(DIM // LANES):
                        outb[t, pl.ds(g * LANES, LANES)] = (
                            rows[t, pl.ds(cb + g * LANES, LANES)] * SCALE
                        )

            def extract_group(q, carry):
                extract_16(q * LANES)
                return carry

            lax.fori_loop(0, SEQ // LANES, extract_group, 0)
            extract_16(SEQ - LANES)

            pltpu.async_copy(outb, out_hbm.at[sbase + c], ssems[b])

        def pair_steps(p, carry):
            step(2 * p, 0)
            step(2 * p + 1, 1)
            return carry

        lax.fori_loop(0, s_per_w // 2, pair_steps, 0)
        wait_store((s_per_w - 1) % 2)

    return body


def kernel(x, table):
    num_seqs = x.shape[0]
    idx = jnp.reshape(x, (x.size,)).astype(jnp.int32)
    idxp = idx >> 1
    cb = (idx & 1) << 6
    pairs = jnp.reshape(table, (table.shape[0] // 2, 2 * DIM))
    return _emb_kernel(num_seqs)(idxp, cb, pairs)


# speedup vs baseline: 1.1078x; 1.0652x over previous
import functools
import math

import jax
import jax.numpy as jnp
from jax import lax
from jax.experimental import pallas as pl
from jax.experimental.pallas import tpu as pltpu
from jax.experimental.pallas import tpu_sc as plsc

DIM = 64
SCALE = math.sqrt(DIM)
NUM_CORES = 2
NUM_SUBCORES = 16
NUM_WORKERS = NUM_CORES * NUM_SUBCORES
LANES = 16

SEQ = 200
GATHER_SUBS = (128, 72)


def _emb_kernel(num_seqs):
    s_per_w = num_seqs // NUM_WORKERS
    t_per_w = s_per_w * SEQ
    mesh = plsc.VectorSubcoreMesh(core_axis_name="c", subcore_axis_name="s")

    @functools.partial(
        pl.kernel,
        mesh=mesh,
        out_type=jax.ShapeDtypeStruct((num_seqs * SEQ, DIM), jnp.float32),
        scratch_types=[
            pltpu.VMEM((t_per_w,), jnp.int32),
            pltpu.VMEM((SEQ,), jnp.int32),
            pltpu.VMEM((SEQ,), jnp.int32),
            pltpu.VMEM((SEQ, 2 * DIM), jnp.float32),
            pltpu.VMEM((SEQ, 2 * DIM), jnp.float32),
            pltpu.VMEM((SEQ, DIM), jnp.float32),
            pltpu.VMEM((SEQ, DIM), jnp.float32),
            pltpu.SemaphoreType.DMA,
            pltpu.SemaphoreType.DMA,
            pltpu.SemaphoreType.DMA,
            pltpu.SemaphoreType.DMA,
        ],
        compiler_params=pltpu.CompilerParams(use_tc_tiling_on_sc=True),
    )
    def body(idxp_hbm, cb_hbm, pairs_hbm, out_hbm,
             idxp_v, cb_a, cb_b, rows_a, rows_b, out_a, out_b,
             gsem_a, gsem_b, ssem_a, ssem_b):
        wid = lax.axis_index("s") * NUM_CORES + lax.axis_index("c")
        sbase = wid * s_per_w
        cbufs = (cb_a, cb_b)
        rbufs = (rows_a, rows_b)
        obufs = (out_a, out_b)
        gsems = (gsem_a, gsem_b)
        ssems = (ssem_a, ssem_b)

        tbase = pl.multiple_of(sbase * SEQ, 8)
        pltpu.sync_copy(idxp_hbm.at[pl.ds(tbase, t_per_w)], idxp_v)

        def fire_gather(c, b):
            off = 0
            for n in GATHER_SUBS:
                pltpu.async_copy(
                    pairs_hbm.at[idxp_v.at[pl.ds(c * SEQ + off, n)]],
                    rbufs[b].at[pl.ds(off, n)],
                    gsems[b],
                )
                off += n

        def drain_gather(b):
            off = 0
            for n in GATHER_SUBS:
                pltpu.make_async_copy(
                    pairs_hbm.at[idxp_v.at[pl.ds(off, n)]],
                    rbufs[b].at[pl.ds(off, n)],
                    gsems[b],
                ).wait()
                off += n

        def load_cb(c, b):
            off = pl.multiple_of((sbase + c) * SEQ, 8)
            pltpu.sync_copy(cb_hbm.at[pl.ds(off, SEQ)], cbufs[b])

        def wait_store(b):
            pltpu.make_async_copy(
                obufs[b], out_hbm.at[pl.ds(0, SEQ)], ssems[b]
            ).wait()

        load_cb(0, 0)
        fire_gather(0, 0)

        def step(c, b):
            rows, outb = rbufs[b], obufs[b]

            @pl.when(c >= 2)
            def _():
                wait_store(b)

            @pl.when(c + 1 < s_per_w)
            def _():
                fire_gather(c + 1, 1 - b)
                load_cb(c + 1, 1 - b)

            drain_gather(b)

            def extract_16(tbase16):
                cb16 = cbufs[b][pl.ds(tbase16, LANES)]
                for j in range(LANES):
                    t = tbase16 + j
                    cb = cb16[j]
                    for g in range(DIM // LANES):
                        outb[t, pl.ds(g * LANES, LANES)] = (
                            rows[t, pl.ds(cb + g * LANES, LANES)] * SCALE
                        )

            def extract_group(q, carry):
                extract_16(q * LANES)
                return carry

            lax.fori_loop(0, SEQ // LANES, extract_group, 0, unroll=2)
            extract_16(SEQ - LANES)

            ooff = pl.multiple_of((sbase + c) * SEQ, 8)
            pltpu.async_copy(outb, out_hbm.at[pl.ds(ooff, SEQ)], ssems[b])

        def pair_steps(p, carry):
            step(2 * p, 0)
            step(2 * p + 1, 1)
            return carry

        lax.fori_loop(0, s_per_w // 2, pair_steps, 0)
        wait_store((s_per_w - 1) % 2)

    return body


def kernel(x, table):
    num_seqs = x.shape[0]
    idx = jnp.reshape(x, (x.size,)).astype(jnp.int32)
    idxp = idx >> 1
    cb = (idx & 1) << 6
    pairs = jnp.reshape(table, (table.shape[0] // 2, 2 * DIM))
    out = _emb_kernel(num_seqs)(idxp, cb, pairs)
    return jnp.reshape(out, x.shape + (DIM,))


# speedup vs baseline: 1.1184x; 1.0095x over previous
import functools
import math

import jax
import jax.numpy as jnp
from jax import lax
from jax.experimental import pallas as pl
from jax.experimental.pallas import tpu as pltpu
from jax.experimental.pallas import tpu_sc as plsc

DIM = 64
SCALE = math.sqrt(DIM)
NUM_CORES = 2
NUM_SUBCORES = 16
NUM_WORKERS = NUM_CORES * NUM_SUBCORES
LANES = 16

SEQ = 200
GATHER_SUBS = (128, 72)


def _emb_kernel(num_seqs):
    s_per_w = num_seqs // NUM_WORKERS
    t_per_w = s_per_w * SEQ
    mesh = plsc.VectorSubcoreMesh(core_axis_name="c", subcore_axis_name="s")

    @functools.partial(
        pl.kernel,
        mesh=mesh,
        out_type=jax.ShapeDtypeStruct((num_seqs * SEQ, DIM), jnp.float32),
        scratch_types=[
            pltpu.VMEM((t_per_w,), jnp.int32),
            pltpu.VMEM((SEQ,), jnp.int32),
            pltpu.VMEM((SEQ,), jnp.int32),
            pltpu.VMEM((SEQ, 2 * DIM), jnp.float32),
            pltpu.VMEM((SEQ, 2 * DIM), jnp.float32),
            pltpu.VMEM((SEQ, DIM), jnp.float32),
            pltpu.VMEM((SEQ, DIM), jnp.float32),
            pltpu.SemaphoreType.DMA,
            pltpu.SemaphoreType.DMA,
            pltpu.SemaphoreType.DMA,
            pltpu.SemaphoreType.DMA,
        ],
        compiler_params=pltpu.CompilerParams(use_tc_tiling_on_sc=True),
    )
    def body(idxp_hbm, cb_hbm, pairs_hbm, out_hbm,
             idxp_v, cb_a, cb_b, rows_a, rows_b, out_a, out_b,
             gsem_a, gsem_b, ssem_a, ssem_b):
        wid = lax.axis_index("s") * NUM_CORES + lax.axis_index("c")
        sbase = wid * s_per_w
        cbufs = (cb_a, cb_b)
        rbufs = (rows_a, rows_b)
        obufs = (out_a, out_b)
        gsems = (gsem_a, gsem_b)
        ssems = (ssem_a, ssem_b)

        tbase = pl.multiple_of(sbase * SEQ, 8)
        pltpu.sync_copy(idxp_hbm.at[pl.ds(tbase, t_per_w)], idxp_v)

        def fire_gather(c, b):
            off = 0
            for n in GATHER_SUBS:
                pltpu.async_copy(
                    pairs_hbm.at[idxp_v.at[pl.ds(c * SEQ + off, n)]],
                    rbufs[b].at[pl.ds(off, n)],
                    gsems[b],
                )
                off += n

        def drain_gather(b):
            off = 0
            for n in GATHER_SUBS:
                pltpu.make_async_copy(
                    pairs_hbm.at[idxp_v.at[pl.ds(off, n)]],
                    rbufs[b].at[pl.ds(off, n)],
                    gsems[b],
                ).wait()
                off += n

        def load_cb(c, b):
            off = pl.multiple_of((sbase + c) * SEQ, 8)
            pltpu.sync_copy(cb_hbm.at[pl.ds(off, SEQ)], cbufs[b])

        def wait_store(b):
            pltpu.make_async_copy(
                obufs[b], out_hbm.at[pl.ds(0, SEQ)], ssems[b]
            ).wait()

        load_cb(0, 0)
        fire_gather(0, 0)

        def step(c, b):
            rows, outb = rbufs[b], obufs[b]

            @pl.when(c >= 2)
            def _():
                wait_store(b)

            @pl.when(c + 1 < s_per_w)
            def _():
                fire_gather(c + 1, 1 - b)
                load_cb(c + 1, 1 - b)

            drain_gather(b)

            def extract_16(tbase16):
                cb16 = cbufs[b][pl.ds(tbase16, LANES)]
                for j in range(LANES):
                    t = tbase16 + j
                    cb = cb16[j]
                    for g in range(DIM // LANES):
                        outb[t, pl.ds(g * LANES, LANES)] = (
                            rows[t, pl.ds(cb + g * LANES, LANES)] * SCALE
                        )

            def extract_group(q, carry):
                extract_16(q * LANES)
                return carry

            lax.fori_loop(0, SEQ // LANES, extract_group, 0)
            extract_16(SEQ - LANES)

            ooff = pl.multiple_of((sbase + c) * SEQ, 8)
            pltpu.async_copy(outb, out_hbm.at[pl.ds(ooff, SEQ)], ssems[b])

        def pair_steps(p, carry):
            step(2 * p, 0)
            step(2 * p + 1, 1)
            return carry

        lax.fori_loop(0, s_per_w // 2, pair_steps, 0)
        wait_store((s_per_w - 1) % 2)

    return body


def kernel(x, table):
    num_seqs = x.shape[0]
    idx = jnp.reshape(x, (x.size,)).astype(jnp.int32)
    idxp = idx >> 1
    cb = (idx & 1) << 6
    pairs = jnp.reshape(table, (table.shape[0] // 2, 2 * DIM))
    out = _emb_kernel(num_seqs)(idxp, cb, pairs)
    return jnp.reshape(out, x.shape + (DIM,))
